# baseline (speedup 1.0000x reference)
import jax
import jax.numpy as jnp
from jax import lax
from jax.experimental import pallas as pl
from jax.experimental.pallas import tpu as pltpu
from jax.experimental.pallas import tpu_sc as plsc

_D = 64
_SCALE = 8.0

_NC = 2
_NS = 16
_NW = _NC * _NS

_BATCH = 4096
_SEQ = 200
_SPLITS = ((0, 104), (104, 96))
_RPW = _BATCH // _NW
_K = 4
_NCHUNK = _RPW // _K


def _body(x_hbm, tab_hbm, out_hbm, idx_v, rows_v, gsem, osem):
    wid = lax.axis_index("s") * _NC + lax.axis_index("c")
    row0 = wid * _RPW

    def fire_gathers(buf):
        for j in range(_K):
            for off, ln in _SPLITS:
                pltpu.async_copy(
                    tab_hbm.at[idx_v.at[buf, j, pl.ds(off, ln)]],
                    rows_v.at[buf, j, pl.ds(off, ln)],
                    gsem,
                )

    def wait_gathers(buf):
        for j in range(_K):
            for off, ln in _SPLITS:
                pltpu.make_async_copy(
                    tab_hbm.at[idx_v.at[buf, j, pl.ds(off, ln)]],
                    rows_v.at[buf, j, pl.ds(off, ln)],
                    gsem,
                ).wait()

    def scatter_desc(g, buf):
        return pltpu.make_async_copy(
            rows_v.at[buf],
            out_hbm.at[pl.ds(row0 + g * _K, _K)],
            osem,
        )

    pltpu.sync_copy(x_hbm.at[pl.ds(row0, _K)], idx_v.at[0])
    fire_gathers(0)

    @pl.loop(0, _NCHUNK, step=2)
    def _pair(g0):
        for phase in range(2):
            g = g0 + phase
            cur, nxt = phase, 1 - phase

            @pl.when(g >= 1)
            def _():
                scatter_desc(g - 1, nxt).wait()

            @pl.when(g + 1 < _NCHUNK)
            def _():
                pltpu.sync_copy(
                    x_hbm.at[pl.ds(row0 + (g + 1) * _K, _K)], idx_v.at[nxt]
                )
                fire_gathers(nxt)

            wait_gathers(cur)

            for j in range(_K):
                @pl.loop(0, _SEQ)
                def _scale(r):
                    for c in range(_D // 16):
                        sl = pl.ds(c * 16, 16)
                        rows_v[cur, j, r, sl] = rows_v[cur, j, r, sl] * _SCALE

            scatter_desc(g, cur).start()

    scatter_desc(_NCHUNK - 1, (_NCHUNK - 1) % 2).wait()


@jax.jit
def _embed(x, table):
    mesh = plsc.VectorSubcoreMesh(
        core_axis_name="c", subcore_axis_name="s",
        num_cores=_NC, num_subcores=_NS,
    )
    f = pl.kernel(
        _body,
        out_type=jax.ShapeDtypeStruct((_BATCH, _SEQ, _D), jnp.float32),
        mesh=mesh,
        scratch_types=[
            pltpu.VMEM((2, _K, _SEQ), jnp.int32),
            pltpu.VMEM((2, _K, _SEQ, _D), jnp.float32),
            pltpu.SemaphoreType.DMA,
            pltpu.SemaphoreType.DMA,
        ],
        compiler_params=pltpu.CompilerParams(use_tc_tiling_on_sc=False),
    )
    return f(x, table)


def kernel(x, embedding):
    return _embed(x, embedding)
